# Initial kernel scaffold; baseline (speedup 1.0000x reference)
import jax
import jax.numpy as jnp
from jax.experimental import pallas as pl


def kernel(batch, table):
    raise NotImplementedError("write your pallas kernel here")


# speedup vs baseline: 1.4593x; 1.4593x over previous
import functools

import jax
import jax.numpy as jnp
from jax import lax
from jax.experimental import pallas as pl
from jax.experimental.pallas import tpu as pltpu
from jax.experimental.pallas import tpu_sc as plsc

_VOCAB = 1000000
_EMB = 64
_L = 50
_RAD = 5
_NEG = 5
_B = 4096

_NC = 2
_NS = 16
_NW = _NC * _NS
_ROWS = _B * _L + _B * _NEG
_RPW = _ROWS // _NW
_CH = 128
_NCH = _RPW // _CH


def _sc_gather(table, idx3):
    mesh = plsc.VectorSubcoreMesh(core_axis_name="c", subcore_axis_name="s")

    @functools.partial(
        pl.kernel,
        mesh=mesh,
        compiler_params=pltpu.CompilerParams(use_tc_tiling_on_sc=False),
        out_type=jax.ShapeDtypeStruct((_ROWS, _EMB), jnp.float32),
        scratch_types=[
            pltpu.VMEM((_NCH, _CH), jnp.int32),
            pltpu.VMEM((_CH, _EMB), jnp.float32),
            pltpu.SemaphoreType.DMA,
        ],
    )
    def gather_kernel(table_hbm, idx_hbm, out_hbm, idx_v, rows_v, sem):
        wid = lax.axis_index("s") * _NC + lax.axis_index("c")
        pltpu.sync_copy(idx_hbm.at[wid], idx_v)
        base = wid * _RPW

        def body(j, carry):
            pltpu.async_copy(table_hbm.at[idx_v.at[j]], rows_v, sem).wait()
            pltpu.sync_copy(rows_v, out_hbm.at[pl.ds(base + j * _CH, _CH)])
            return carry

        lax.fori_loop(0, _NCH, body, 0)

    return gather_kernel(table, idx3)


def _tc_loss(gathered):
    bb = 128
    grid = _B // bb
    pos_rows = bb * _L
    neg_rows = bb * _NEG
    neg_block0 = (_B * _L) // neg_rows

    def body(pos_ref, neg_ref, out_ref):
        i = pl.program_id(0)
        E = pos_ref[...].reshape(bb, _L, _EMB)
        N = neg_ref[...].reshape(bb, _NEG, _EMB)

        pos_sum = jnp.float32(0.0)
        for k in range(1, _RAD + 1):
            s = jnp.sum(E[:, : _L - k, :] * E[:, k:, :], axis=2)
            p = jax.nn.sigmoid(s)
            f = jnp.where(p > 0, -jnp.log(jnp.where(p > 0, p, 1.0)), 100.0)
            pos_sum += 2.0 * jnp.sum(f)

        neg_sum = jnp.float32(0.0)
        for n in range(_NEG):
            s = jnp.sum(E * N[:, n : n + 1, :], axis=2)
            p = jax.nn.sigmoid(s)
            q = 1.0 - p
            g = jnp.where(q > 0, -jnp.log(jnp.where(q > 0, q, 1.0)), 100.0)
            neg_sum += jnp.sum(g)

        lanes = lax.broadcasted_iota(jnp.int32, (1, 128), 1)
        vec = (jnp.where(lanes == 0, pos_sum, 0.0)
               + jnp.where(lanes == 1, neg_sum, 0.0))

        @pl.when(i == 0)
        def _():
            out_ref[...] = jnp.zeros_like(out_ref)

        out_ref[...] += vec

    out = pl.pallas_call(
        body,
        grid=(grid,),
        in_specs=[
            pl.BlockSpec((pos_rows, _EMB), lambda i: (i, 0)),
            pl.BlockSpec((neg_rows, _EMB), lambda i: (i + neg_block0, 0)),
        ],
        out_specs=pl.BlockSpec((1, 128), lambda i: (0, 0)),
        out_shape=jax.ShapeDtypeStruct((1, 128), jnp.float32),
    )(gathered, gathered)
    return out[0, 0] / (_B * _L * _L) + out[0, 1] / (_B * _L * _NEG)


def kernel(batch, table):
    neg_words = jax.random.randint(
        jax.random.key(1), (_B, _NEG), 1, _VOCAB, dtype=jnp.int32)
    idx = jnp.concatenate([batch.reshape(-1), neg_words.reshape(-1)])
    idx3 = idx.reshape(_NW, _NCH, _CH)
    gathered = _sc_gather(table, idx3)
    return _tc_loss(gathered)


# speedup vs baseline: 2.9370x; 2.0126x over previous
import functools

import jax
import jax.numpy as jnp
import numpy as np
from jax import lax
from jax.experimental import pallas as pl
from jax.experimental.pallas import tpu as pltpu
from jax.experimental.pallas import tpu_sc as plsc

_VOCAB = 1000000
_EMB = 64
_L = 50
_RAD = 5
_NEG = 5
_B = 4096

_NC = 2
_NS = 16
_NW = _NC * _NS
_POS_ROWS = _B * _L
_NEG_ROWS = 2 * _B * _NEG
_ROWS = _POS_ROWS + _NEG_ROWS
_RPW = _ROWS // _NW
_CH = 128
_NCH = _RPW // _CH

_T = _L // 2
_BSUB = 2048
_NB2 = _B // _BSUB
_NPROD = 11
_NCOL = 2 * _NPROD


def _sc_gather(table, idx3):
    mesh = plsc.VectorSubcoreMesh(core_axis_name="c", subcore_axis_name="s")

    @functools.partial(
        pl.kernel,
        mesh=mesh,
        compiler_params=pltpu.CompilerParams(use_tc_tiling_on_sc=False),
        out_type=jax.ShapeDtypeStruct((_ROWS, _EMB), jnp.float32),
        scratch_types=[
            pltpu.VMEM((_NCH, _CH), jnp.int32),
            pltpu.VMEM((_CH, _EMB), jnp.float32),
            pltpu.SemaphoreType.DMA,
        ],
    )
    def gather_kernel(table_hbm, idx_hbm, out_hbm, idx_v, rows_v, sem):
        wid = lax.axis_index("s") * _NC + lax.axis_index("c")
        pltpu.sync_copy(idx_hbm.at[wid], idx_v)
        base = wid * _RPW

        def body(j, carry):
            pltpu.async_copy(table_hbm.at[idx_v.at[j]], rows_v, sem).wait()
            pltpu.sync_copy(rows_v, out_hbm.at[pl.ds(base + j * _CH, _CH)])
            return carry

        lax.fori_loop(0, _NCH, body, 0)

    return gather_kernel(table, idx3)


def _sel_matrix():
    sel = np.zeros((_NPROD * 128, 128), np.float32)
    for p in range(_NPROD):
        sel[p * 128: p * 128 + 64, 2 * p] = 1.0
        sel[p * 128 + 64: (p + 1) * 128, 2 * p + 1] = 1.0
    return jnp.asarray(sel)


_POS_TMAX = [23, 23, 22, 22, 24, -1, 23, 23, 22, 22, -1, 21]


def _tc_loss(g2):
    nblk = _ROWS // 2 // _BSUB
    negblk0 = _POS_ROWS // 2 // _BSUB

    def body(a_ref, b1_ref, b2_ref, b3_ref, n0, n1, n2, n3, n4, sel_ref,
             out_ref, s_ref):
        i2 = pl.program_id(0)
        t = pl.program_id(1)
        a = a_ref[...]
        prods = [
            a * b1_ref[...],
            a * b2_ref[...],
            a * pltpu.roll(a, 64, 1),
            a * pltpu.roll(b1_ref[...], 64, 1),
            a * pltpu.roll(b2_ref[...], 64, 1),
            a * pltpu.roll(b3_ref[...], 64, 1),
            a * n0[...],
            a * n1[...],
            a * n2[...],
            a * n3[...],
            a * n4[...],
        ]
        for p in range(_NPROD):
            s_ref[:, p * 128:(p + 1) * 128] = prods[p]
        sims = jnp.dot(s_ref[...], sel_ref[...],
                       preferred_element_type=jnp.float32)

        p_ = jax.nn.sigmoid(sims)
        f = jnp.where(p_ > 0, -jnp.log(jnp.where(p_ > 0, p_, 1.0)), 100.0)
        q_ = 1.0 - p_
        g = jnp.where(q_ > 0, -jnp.log(jnp.where(q_ > 0, q_, 1.0)), 100.0)

        lanes = lax.broadcasted_iota(jnp.int32, (1, 128), 1)
        tmax = jnp.full((1, 128), -1, jnp.int32)
        for c, tm in enumerate(_POS_TMAX):
            tmax = jnp.where(lanes == c, tm, tmax)
        is_pos = lanes < 12
        is_neg = (lanes >= 12) & (lanes < _NCOL)
        w = jnp.where(is_pos & (t <= tmax), 2.0,
                      jnp.where(is_neg, 1.0, 0.0))
        vals = jnp.where(is_pos, f, g) * w
        part = jnp.sum(vals, axis=0, keepdims=True)

        @pl.when((i2 == 0) & (t == 0))
        def _():
            out_ref[...] = jnp.zeros_like(out_ref)

        out_ref[...] += part

    bspec = lambda im: pl.BlockSpec((_BSUB, 128), im)
    out = pl.pallas_call(
        body,
        grid=(_NB2, _T),
        in_specs=[
            bspec(lambda i2, t: (t * _NB2 + i2, 0)),
            bspec(lambda i2, t: (jnp.minimum(t + 1, _T - 1) * _NB2 + i2, 0)),
            bspec(lambda i2, t: (jnp.minimum(t + 2, _T - 1) * _NB2 + i2, 0)),
            bspec(lambda i2, t: (jnp.minimum(t + 3, _T - 1) * _NB2 + i2, 0)),
            bspec(lambda i2, t: (negblk0 + 0 * _NB2 + i2, 0)),
            bspec(lambda i2, t: (negblk0 + 1 * _NB2 + i2, 0)),
            bspec(lambda i2, t: (negblk0 + 2 * _NB2 + i2, 0)),
            bspec(lambda i2, t: (negblk0 + 3 * _NB2 + i2, 0)),
            bspec(lambda i2, t: (negblk0 + 4 * _NB2 + i2, 0)),
            pl.BlockSpec((_NPROD * 128, 128), lambda i2, t: (0, 0)),
        ],
        out_specs=pl.BlockSpec((1, 128), lambda i2, t: (0, 0)),
        out_shape=jax.ShapeDtypeStruct((1, 128), jnp.float32),
        scratch_shapes=[pltpu.VMEM((_BSUB, _NPROD * 128), jnp.float32)],
    )(g2, g2, g2, g2, g2, g2, g2, g2, g2, _sel_matrix())
    pos_sum = jnp.sum(out[0, :12])
    neg_sum = jnp.sum(out[0, 12:_NCOL])
    return pos_sum / (_B * _L * _L) + neg_sum / (_B * _L * _NEG)


def kernel(batch, table):
    neg_words = jax.random.randint(
        jax.random.key(1), (_B, _NEG), 1, _VOCAB, dtype=jnp.int32)
    pos_idx = (batch.T.reshape(_T, 2, _B)
               .transpose(0, 2, 1).reshape(-1))
    neg_idx = jnp.broadcast_to(
        neg_words.T.reshape(_NEG, _B, 1), (_NEG, _B, 2)).reshape(-1)
    idx = jnp.concatenate([pos_idx, neg_idx])
    idx3 = idx.reshape(_NW, _NCH, _CH)
    gathered = _sc_gather(table, idx3)
    g2 = gathered.reshape(_ROWS // 2, 128)
    return _tc_loss(g2)
